# baseline (speedup 1.0000x reference)
import functools

import jax
import jax.numpy as jnp
from jax import lax
from jax.experimental import pallas as pl
from jax.experimental.pallas import tpu as pltpu
from jax.experimental.pallas import tpu_sc as plsc

B = 4096
L = 200
D = 64
NUM_CLASSES = 10
NW = 32
RPW = B // NW
NCHUNK = 2
CHUNK = L // NCHUNK
NLANE = D // 16


def _sc_pool(inputs3, table):
    mesh = plsc.VectorSubcoreMesh(core_axis_name="c", subcore_axis_name="s")

    @functools.partial(
        pl.kernel,
        mesh=mesh,
        out_type=[
            jax.ShapeDtypeStruct((B, D), jnp.float32),
            jax.ShapeDtypeStruct((B, D), jnp.float32),
        ],
        scratch_types=[
            pltpu.VMEM((2, NCHUNK, CHUNK), jnp.int32),
            pltpu.VMEM((2, L, D), jnp.float32),
            pltpu.VMEM((RPW, D), jnp.float32),
            pltpu.VMEM((RPW, D), jnp.float32),
            pltpu.SemaphoreType.DMA,
            pltpu.SemaphoreType.DMA,
        ],
        compiler_params=pltpu.CompilerParams(use_tc_tiling_on_sc=False),
    )
    def k(inputs_hbm, table_hbm, avg_hbm, max_hbm,
          idx_v, rows_v, avg_buf, max_buf, sem0, sem1):
        cid = lax.axis_index("c")
        sid = lax.axis_index("s")
        wid = sid * 2 + cid
        base = wid * RPW
        sems = (sem0, sem1)

        def issue(row, slot):
            pltpu.sync_copy(inputs_hbm.at[base + row], idx_v.at[slot])
            for j in range(NCHUNK):
                pltpu.async_copy(
                    table_hbm.at[idx_v.at[slot, j]],
                    rows_v.at[slot, pl.ds(j * CHUNK, CHUNK)],
                    sems[slot],
                )

        def wait_slot(slot):
            for j in range(NCHUNK):
                pltpu.make_async_copy(
                    table_hbm.at[idx_v.at[slot, j]],
                    rows_v.at[slot, pl.ds(j * CHUNK, CHUNK)],
                    sems[slot],
                ).wait()

        def reduce_store(row, slot):
            def body(i, carry):
                out = []
                for d in range(NLANE):
                    v = rows_v[slot, i, pl.ds(d * 16, 16)]
                    out.append(carry[2 * d] + v)
                    out.append(jnp.maximum(carry[2 * d + 1], v))
                return tuple(out)

            init = []
            for _ in range(NLANE):
                init.append(jnp.zeros((16,), jnp.float32))
                init.append(jnp.full((16,), -jnp.inf, jnp.float32))
            res = lax.fori_loop(0, L, body, tuple(init))
            for d in range(NLANE):
                avg_buf[row, pl.ds(d * 16, 16)] = res[2 * d] * (1.0 / L)
                max_buf[row, pl.ds(d * 16, 16)] = res[2 * d + 1]

        issue(0, 0)

        def outer(g, carry):
            for b in range(2):
                row = g * 2 + b

                @pl.when(row + 1 < RPW)
                def _():
                    issue(row + 1, 1 - b)

                wait_slot(b)
                reduce_store(row, b)
            return carry

        lax.fori_loop(0, RPW // 2, outer, 0)

        pltpu.sync_copy(avg_buf, avg_hbm.at[pl.ds(base, RPW)])
        pltpu.sync_copy(max_buf, max_hbm.at[pl.ds(base, RPW)])

    return k(inputs3, table)


def _head_body(avg_ref, max_ref, w1_ref, w2_ref, b_ref, out_ref):
    logits = (
        jnp.dot(avg_ref[...], w1_ref[...], preferred_element_type=jnp.float32)
        + jnp.dot(max_ref[...], w2_ref[...], preferred_element_type=jnp.float32)
        + b_ref[...]
    )
    m = jnp.max(logits, axis=-1, keepdims=True)
    e = jnp.exp(logits - m)
    out_ref[...] = e / jnp.sum(e, axis=-1, keepdims=True)


def _tc_head(avg, mx, fc_w, fc_b):
    w1 = fc_w[:D]
    w2 = fc_w[D:]
    b2 = fc_b.reshape(1, NUM_CLASSES)
    return pl.pallas_call(
        _head_body,
        out_shape=jax.ShapeDtypeStruct((B, NUM_CLASSES), jnp.float32),
    )(avg, mx, w1, w2, b2)


def kernel(inputs, table, fc_w, fc_b):
    inputs3 = inputs.astype(jnp.int32).reshape(B, NCHUNK, CHUNK)
    avg, mx = _sc_pool(inputs3, table)
    return _tc_head(avg, mx, fc_w, fc_b)


# speedup vs baseline: 1.0641x; 1.0641x over previous
import functools

import jax
import jax.numpy as jnp
from jax import lax
from jax.experimental import pallas as pl
from jax.experimental.pallas import tpu as pltpu
from jax.experimental.pallas import tpu_sc as plsc

B = 4096
L = 200
D = 64
DP = 128
NUM_CLASSES = 10
NW = 32
RPW = B // NW
CHUNKS = (104, 96)
NLANE = D // 16


def _sc_pool(idx1, table128):
    mesh = plsc.VectorSubcoreMesh(core_axis_name="c", subcore_axis_name="s")

    @functools.partial(
        pl.kernel,
        mesh=mesh,
        out_type=jax.ShapeDtypeStruct((B, DP), jnp.float32),
        scratch_types=[
            pltpu.VMEM((RPW * L,), jnp.int32),
            pltpu.VMEM((2, L, DP), jnp.float32),
            pltpu.VMEM((RPW, DP), jnp.float32),
            pltpu.SemaphoreType.DMA,
            pltpu.SemaphoreType.DMA,
        ],
        compiler_params=pltpu.CompilerParams(needs_layout_passes=False),
    )
    def k(idx_hbm, table_hbm, cat_hbm, idx_v, rows_v, cat_buf, sem0, sem1):
        cid = lax.axis_index("c")
        sid = lax.axis_index("s")
        wid = sid * 2 + cid
        base = wid * RPW
        sems = (sem0, sem1)
        lanes = lax.iota(jnp.int32, 16)

        def issue(row, slot):
            off = 0
            for c in CHUNKS:
                pltpu.async_copy(
                    table_hbm.at[idx_v.at[pl.ds(row * L + off, c)]],
                    rows_v.at[slot, pl.ds(off, c)],
                    sems[slot],
                )
                off += c

        def wait_slot(slot):
            off = 0
            for c in CHUNKS:
                pltpu.make_async_copy(
                    table_hbm.at[idx_v.at[pl.ds(off, c)]],
                    rows_v.at[slot, pl.ds(off, c)],
                    sems[slot],
                ).wait()
                off += c

        def reduce_store(row, slot):
            rows2d = rows_v.at[slot]

            def body(i, carry):
                ivec = jnp.full((16,), i, jnp.int32)
                out = []
                for d in range(NLANE):
                    v = plsc.load_gather(rows2d, [ivec, d * 16 + lanes])
                    out.append(carry[2 * d] + v)
                    out.append(jnp.maximum(carry[2 * d + 1], v))
                return tuple(out)

            init = []
            for _ in range(NLANE):
                init.append(jnp.zeros((16,), jnp.float32))
                init.append(jnp.full((16,), -jnp.inf, jnp.float32))
            res = lax.fori_loop(0, L, body, tuple(init))
            rvec = jnp.full((16,), row, jnp.int32)
            for d in range(NLANE):
                plsc.store_scatter(cat_buf, [rvec, d * 16 + lanes],
                                   res[2 * d] * (1.0 / L))
                plsc.store_scatter(cat_buf, [rvec, D + d * 16 + lanes],
                                   res[2 * d + 1])

        pltpu.sync_copy(idx_hbm.at[pl.ds(base * L, RPW * L)], idx_v)
        issue(0, 0)

        def outer(g, carry):
            for b in range(2):
                row = g * 2 + b

                @pl.when(row + 1 < RPW)
                def _():
                    issue(row + 1, 1 - b)

                wait_slot(b)
                reduce_store(row, b)
            return carry

        lax.fori_loop(0, RPW // 2, outer, 0)

        pltpu.sync_copy(cat_buf, cat_hbm.at[pl.ds(base, RPW)])

    return k(idx1, table128)


def _head_body(cat_ref, w_ref, b_ref, out_ref):
    logits = (
        jnp.dot(cat_ref[...], w_ref[...], preferred_element_type=jnp.float32)
        + b_ref[...]
    )
    m = jnp.max(logits, axis=-1, keepdims=True)
    e = jnp.exp(logits - m)
    out_ref[...] = e / jnp.sum(e, axis=-1, keepdims=True)


def _tc_head(cat, fc_w, fc_b):
    b2 = fc_b.reshape(1, NUM_CLASSES)
    return pl.pallas_call(
        _head_body,
        out_shape=jax.ShapeDtypeStruct((B, NUM_CLASSES), jnp.float32),
    )(cat, fc_w, b2)


def kernel(inputs, table, fc_w, fc_b):
    idx1 = inputs.astype(jnp.int32).reshape(B * L)
    table128 = jnp.pad(table, ((0, 0), (0, DP - D)))
    cat = _sc_pool(idx1, table128)
    return _tc_head(cat, fc_w, fc_b)
